# baseline (speedup 1.0000x reference)
import jax
import jax.numpy as jnp
from jax.experimental import pallas as pl
from jax.experimental.pallas import tpu as pltpu

_MIN_DISTANCE = 3
_THRESHOLD_REL = 0.01
_MAX_NUM_PEAKS = 100
_H = 512
_W = 512
_C = 10
_BIG = 1 << 30
_T = 4


def _pool7(padded):
    n = padded.shape[1] - 6
    t2 = jnp.maximum(padded[:, :n + 5], padded[:, 1:])
    t4 = jnp.maximum(t2[:, :n + 3], t2[:, 2:])
    return jnp.maximum(t4[:, :n], t4[:, 3:])


def _nms_kernel(x_ref, scores_ref, gidx_ref, candT_ref):
    x = x_ref[...]

    m = jnp.max(x, axis=(1, 2), keepdims=True)
    e = jnp.exp(x - m)
    s = jnp.sum(e, axis=(1, 2), keepdims=True)
    p = e / s

    zr = jnp.zeros((_C, _MIN_DISTANCE, _W), jnp.float32)
    pooled_v = _pool7(jnp.concatenate([zr, p, zr], axis=1))
    pT = jnp.swapaxes(p, 1, 2)
    pvT = jnp.swapaxes(pooled_v, 1, 2)
    zc = jnp.zeros((_C, _MIN_DISTANCE, _H), jnp.float32)
    pooledT = _pool7(jnp.concatenate([zc, pvT, zc], axis=1))

    thr_abs = 1.0 / (_H * _W) * 2.0
    mx = jnp.max(pT, axis=(1, 2), keepdims=True)
    maskT = (pooledT == pT) & (pT > thr_abs) & (pT > _THRESHOLD_REL * mx)
    candT = jnp.where(maskT, pT, jnp.float32(-1.0))
    candT_ref[...] = candT

    rowsT = jax.lax.broadcasted_iota(jnp.int32, (_C, _W, _H), 2)
    levels = []
    cT = candT
    for _ in range(_T):
        smax = jnp.max(cT, axis=2)
        sarg = jnp.min(jnp.where(cT == smax[:, :, None], rowsT, _BIG),
                       axis=2)
        cT = jnp.where(rowsT == sarg[:, :, None], jnp.float32(-3.0), cT)
        levels.append((smax, sarg))
    candT_ref[...] = cT
    (cmax, carg), (s1m, s1a), (s2m, s2a), (s3m, s3a) = levels

    lane_w = jax.lax.broadcasted_iota(jnp.int32, (_C, _W), 1)
    lane_h = jax.lax.broadcasted_iota(jnp.int32, (_C, _H), 1)
    lane_k = jax.lax.broadcasted_iota(jnp.int32, (_C, 128), 1)

    def body(i, st):
        cmax, carg, s1m, s1a, s2m, s2a, s3m, s3a, svec, gvec = st
        mval = jnp.max(cmax, axis=1, keepdims=True)
        g = jnp.min(jnp.where(cmax == mval, carg * _W + lane_w, _BIG),
                    axis=1, keepdims=True)
        svec = jnp.where(lane_k == i, mval, svec)
        gvec = jnp.where(lane_k == i, g, gvec)
        hit = lane_w == g % _W
        cmax = jnp.where(hit, s1m, cmax)
        carg = jnp.where(hit, s1a, carg)
        s1m = jnp.where(hit, s2m, s1m)
        s1a = jnp.where(hit, s2a, s1a)
        s2m = jnp.where(hit, s3m, s2m)
        s2a = jnp.where(hit, s3a, s2a)
        rowsbuf = jnp.concatenate(
            [candT_ref[pl.ds(ch, 1), pl.ds(g[ch, 0] % _W, 1), :]
             .reshape(1, _H) for ch in range(_C)], axis=0)
        nm = jnp.max(rowsbuf, axis=1, keepdims=True)
        na = jnp.min(jnp.where(rowsbuf == nm, lane_h, _BIG),
                     axis=1, keepdims=True)
        rowsbuf = jnp.where(lane_h == na, jnp.float32(-3.0), rowsbuf)
        for ch in range(_C):
            candT_ref[pl.ds(ch, 1), pl.ds(g[ch, 0] % _W, 1), :] = (
                rowsbuf[ch:ch + 1].reshape(1, 1, _H))
        s3m = jnp.where(hit, nm, s3m)
        s3a = jnp.where(hit, na, s3a)
        return cmax, carg, s1m, s1a, s2m, s2a, s3m, s3a, svec, gvec

    svec0 = jnp.zeros((_C, 128), jnp.float32)
    gvec0 = jnp.zeros((_C, 128), jnp.int32)
    st = (cmax, carg, s1m, s1a, s2m, s2a, s3m, s3a, svec0, gvec0)
    st = jax.lax.fori_loop(0, _MAX_NUM_PEAKS, body, st)
    svec, gvec = st[8], st[9]

    scores_ref[0] = svec
    gidx_ref[0] = gvec


def kernel(belive_map):
    B, S, H, W = belive_map.shape
    bs = B * S
    nprog = bs // _C
    xflat = belive_map.reshape(bs, H, W)
    raw_scores, raw_gidx = pl.pallas_call(
        _nms_kernel,
        grid=(nprog,),
        in_specs=[pl.BlockSpec((_C, H, W), lambda i: (i, 0, 0))],
        out_specs=[
            pl.BlockSpec((1, _C, 128), lambda i: (i, 0, 0)),
            pl.BlockSpec((1, _C, 128), lambda i: (i, 0, 0)),
        ],
        out_shape=[
            jax.ShapeDtypeStruct((nprog, _C, 128), jnp.float32),
            jax.ShapeDtypeStruct((nprog, _C, 128), jnp.int32),
        ],
        scratch_shapes=[pltpu.VMEM((_C, W, H), jnp.float32)],
        compiler_params=pltpu.CompilerParams(
            dimension_semantics=("arbitrary",)),
    )(xflat)

    scores_raw = raw_scores.reshape(bs, 128)[:, :_MAX_NUM_PEAKS]
    scores_raw = scores_raw.reshape(B, S, _MAX_NUM_PEAKS)
    g = raw_gidx.reshape(bs, 128)[:, :_MAX_NUM_PEAKS].reshape(B, S,
                                                              _MAX_NUM_PEAKS)
    valid = scores_raw > 0.0
    scores = jnp.where(valid, scores_raw, 0.0)
    rows = g // W
    cols = g % W
    seg = jnp.broadcast_to(jnp.arange(S, dtype=jnp.int32)[None, :, None],
                           (B, S, _MAX_NUM_PEAKS))
    skeletons = jnp.stack([seg, cols, rows], axis=-1)
    return skeletons, scores, valid


# speedup vs baseline: 1.3697x; 1.3697x over previous
import jax
import jax.numpy as jnp
from jax.experimental import pallas as pl
from jax.experimental.pallas import tpu as pltpu

_MIN_DISTANCE = 3
_THRESHOLD_REL = 0.01
_MAX_NUM_PEAKS = 100
_H = 512
_W = 512
_C = 10
_BIG = 1 << 30
_T = 4


def _pool7(padded):
    n = padded.shape[1] - 6
    t2 = jnp.maximum(padded[:, :n + 5], padded[:, 1:])
    t4 = jnp.maximum(t2[:, :n + 3], t2[:, 2:])
    return jnp.maximum(t4[:, :n], t4[:, 3:])


def _nms_kernel(x_ref, scores_ref, gidx_ref, candT_ref):
    x = x_ref[...]

    m = jnp.max(x, axis=(1, 2), keepdims=True)
    e = jnp.exp(x - m)
    s = jnp.sum(e, axis=(1, 2), keepdims=True)
    p = e / s

    zr = jnp.zeros((_C, _MIN_DISTANCE, _W), jnp.float32)
    pooled_v = _pool7(jnp.concatenate([zr, p, zr], axis=1))
    pT = jnp.swapaxes(p, 1, 2)
    pvT = jnp.swapaxes(pooled_v, 1, 2)
    zc = jnp.zeros((_C, _MIN_DISTANCE, _H), jnp.float32)
    pooledT = _pool7(jnp.concatenate([zc, pvT, zc], axis=1))

    thr_abs = 1.0 / (_H * _W) * 2.0
    mx = jnp.max(pT, axis=(1, 2), keepdims=True)
    maskT = (pooledT == pT) & (pT > thr_abs) & (pT > _THRESHOLD_REL * mx)
    candT = jnp.where(maskT, pT, jnp.float32(-1.0))
    candT_ref[...] = candT

    rowsT = jax.lax.broadcasted_iota(jnp.int32, (_C, _W, _H), 2)
    levels = []
    cT = candT
    for _ in range(_T):
        smax = jnp.max(cT, axis=2)
        sarg = jnp.min(jnp.where(cT == smax[:, :, None], rowsT, _BIG),
                       axis=2)
        cT = jnp.where(rowsT == sarg[:, :, None], jnp.float32(-3.0), cT)
        levels.append((smax, sarg))
    candT_ref[...] = cT
    (cmax, carg), (s1m, s1a), (s2m, s2a), (s3m, s3a) = levels

    lane_w = jax.lax.broadcasted_iota(jnp.int32, (_C, _W), 1)
    lane_h = jax.lax.broadcasted_iota(jnp.int32, (_C, _H), 1)
    lane_k = jax.lax.broadcasted_iota(jnp.int32, (_C, 128), 1)

    def body(i, st):
        cmax, carg, s1m, s1a, s2m, s2a, s3m, s3a, svec, gvec = st
        mval = jnp.max(cmax, axis=1, keepdims=True)
        g = jnp.min(jnp.where(cmax == mval, carg * _W + lane_w, _BIG),
                    axis=1, keepdims=True)
        svec = jnp.where(lane_k == i, mval, svec)
        gvec = jnp.where(lane_k == i, g, gvec)
        hit = lane_w == g % _W
        cmax = jnp.where(hit, s1m, cmax)
        carg = jnp.where(hit, s1a, carg)
        s1m = jnp.where(hit, s2m, s1m)
        s1a = jnp.where(hit, s2a, s1a)
        s2m = jnp.where(hit, s3m, s2m)
        s2a = jnp.where(hit, s3a, s2a)
        rowsbuf = jnp.concatenate(
            [candT_ref[pl.ds(ch, 1), pl.ds(g[ch, 0] % _W, 1), :]
             .reshape(1, _H) for ch in range(_C)], axis=0)
        nm = jnp.max(rowsbuf, axis=1, keepdims=True)
        na = jnp.min(jnp.where(rowsbuf == nm, lane_h, _BIG),
                     axis=1, keepdims=True)
        rowsbuf = jnp.where(lane_h == na, jnp.float32(-3.0), rowsbuf)
        for ch in range(_C):
            candT_ref[pl.ds(ch, 1), pl.ds(g[ch, 0] % _W, 1), :] = (
                rowsbuf[ch:ch + 1].reshape(1, 1, _H))
        s3m = jnp.where(hit, nm, s3m)
        s3a = jnp.where(hit, na, s3a)
        return cmax, carg, s1m, s1a, s2m, s2a, s3m, s3a, svec, gvec

    svec0 = jnp.zeros((_C, 128), jnp.float32)
    gvec0 = jnp.zeros((_C, 128), jnp.int32)
    st = (cmax, carg, s1m, s1a, s2m, s2a, s3m, s3a, svec0, gvec0)
    st = jax.lax.fori_loop(0, _MAX_NUM_PEAKS, body, st, unroll=4)
    svec, gvec = st[8], st[9]

    scores_ref[0] = svec
    gidx_ref[0] = gvec


def kernel(belive_map):
    B, S, H, W = belive_map.shape
    bs = B * S
    nprog = bs // _C
    xflat = belive_map.reshape(bs, H, W)
    raw_scores, raw_gidx = pl.pallas_call(
        _nms_kernel,
        grid=(nprog,),
        in_specs=[pl.BlockSpec((_C, H, W), lambda i: (i, 0, 0))],
        out_specs=[
            pl.BlockSpec((1, _C, 128), lambda i: (i, 0, 0)),
            pl.BlockSpec((1, _C, 128), lambda i: (i, 0, 0)),
        ],
        out_shape=[
            jax.ShapeDtypeStruct((nprog, _C, 128), jnp.float32),
            jax.ShapeDtypeStruct((nprog, _C, 128), jnp.int32),
        ],
        scratch_shapes=[pltpu.VMEM((_C, W, H), jnp.float32)],
        compiler_params=pltpu.CompilerParams(
            dimension_semantics=("arbitrary",)),
    )(xflat)

    scores_raw = raw_scores.reshape(bs, 128)[:, :_MAX_NUM_PEAKS]
    scores_raw = scores_raw.reshape(B, S, _MAX_NUM_PEAKS)
    g = raw_gidx.reshape(bs, 128)[:, :_MAX_NUM_PEAKS].reshape(B, S,
                                                              _MAX_NUM_PEAKS)
    valid = scores_raw > 0.0
    scores = jnp.where(valid, scores_raw, 0.0)
    rows = g // W
    cols = g % W
    seg = jnp.broadcast_to(jnp.arange(S, dtype=jnp.int32)[None, :, None],
                           (B, S, _MAX_NUM_PEAKS))
    skeletons = jnp.stack([seg, cols, rows], axis=-1)
    return skeletons, scores, valid


# speedup vs baseline: 1.4716x; 1.0744x over previous
import jax
import jax.numpy as jnp
from jax.experimental import pallas as pl
from jax.experimental.pallas import tpu as pltpu

_MIN_DISTANCE = 3
_THRESHOLD_REL = 0.01
_MAX_NUM_PEAKS = 100
_H = 512
_W = 512
_C = 10
_BIG = 1 << 30
_T = 4


def _pool7(padded):
    n = padded.shape[1] - 6
    t2 = jnp.maximum(padded[:, :n + 5], padded[:, 1:])
    t4 = jnp.maximum(t2[:, :n + 3], t2[:, 2:])
    return jnp.maximum(t4[:, :n], t4[:, 3:])


def _nms_kernel(x_ref, scores_ref, gidx_ref, candT_ref):
    x = x_ref[...]

    m = jnp.max(x, axis=(1, 2), keepdims=True)
    e = jnp.exp(x - m)
    s = jnp.sum(e, axis=(1, 2), keepdims=True)
    p = e / s

    zr = jnp.zeros((_C, _MIN_DISTANCE, _W), jnp.float32)
    pooled_v = _pool7(jnp.concatenate([zr, p, zr], axis=1))
    pT = jnp.swapaxes(p, 1, 2)
    pvT = jnp.swapaxes(pooled_v, 1, 2)
    zc = jnp.zeros((_C, _MIN_DISTANCE, _H), jnp.float32)
    pooledT = _pool7(jnp.concatenate([zc, pvT, zc], axis=1))

    thr_abs = 1.0 / (_H * _W) * 2.0
    mx = jnp.max(pT, axis=(1, 2), keepdims=True)
    maskT = (pooledT == pT) & (pT > thr_abs) & (pT > _THRESHOLD_REL * mx)
    candT = jnp.where(maskT, pT, jnp.float32(-1.0))
    candT_ref[...] = candT

    rowsT = jax.lax.broadcasted_iota(jnp.int32, (_C, _W, _H), 2)
    levels = []
    cT = candT
    for _ in range(_T):
        smax = jnp.max(cT, axis=2)
        sarg = jnp.min(jnp.where(cT == smax[:, :, None], rowsT, _BIG),
                       axis=2)
        cT = jnp.where(rowsT == sarg[:, :, None], jnp.float32(-3.0), cT)
        levels.append((smax, sarg))
    candT_ref[...] = cT
    (cmax, carg), (s1m, s1a), (s2m, s2a), (s3m, s3a) = levels

    lane_w = jax.lax.broadcasted_iota(jnp.int32, (_C, _W), 1)
    lane_h = jax.lax.broadcasted_iota(jnp.int32, (_C, _H), 1)
    lane_k = jax.lax.broadcasted_iota(jnp.int32, (_C, 128), 1)

    def body(i, st):
        cmax, carg, s1m, s1a, s2m, s2a, s3m, s3a, svec, gvec = st
        mval = jnp.max(cmax, axis=1, keepdims=True)
        g = jnp.min(jnp.where(cmax == mval, carg * _W + lane_w, _BIG),
                    axis=1, keepdims=True)
        svec = jnp.where(lane_k == i, mval, svec)
        gvec = jnp.where(lane_k == i, g, gvec)
        hit = lane_w == g % _W
        cmax = jnp.where(hit, s1m, cmax)
        carg = jnp.where(hit, s1a, carg)
        s1m = jnp.where(hit, s2m, s1m)
        s1a = jnp.where(hit, s2a, s1a)
        s2m = jnp.where(hit, s3m, s2m)
        s2a = jnp.where(hit, s3a, s2a)
        rowsbuf = jnp.concatenate(
            [candT_ref[pl.ds(ch, 1), pl.ds(g[ch, 0] % _W, 1), :]
             .reshape(1, _H) for ch in range(_C)], axis=0)
        nm = jnp.max(rowsbuf, axis=1, keepdims=True)
        na = jnp.min(jnp.where(rowsbuf == nm, lane_h, _BIG),
                     axis=1, keepdims=True)
        rowsbuf = jnp.where(lane_h == na, jnp.float32(-3.0), rowsbuf)
        for ch in range(_C):
            candT_ref[pl.ds(ch, 1), pl.ds(g[ch, 0] % _W, 1), :] = (
                rowsbuf[ch:ch + 1].reshape(1, 1, _H))
        s3m = jnp.where(hit, nm, s3m)
        s3a = jnp.where(hit, na, s3a)
        return cmax, carg, s1m, s1a, s2m, s2a, s3m, s3a, svec, gvec

    svec0 = jnp.zeros((_C, 128), jnp.float32)
    gvec0 = jnp.zeros((_C, 128), jnp.int32)
    st = (cmax, carg, s1m, s1a, s2m, s2a, s3m, s3a, svec0, gvec0)
    st = jax.lax.fori_loop(0, _MAX_NUM_PEAKS, body, st, unroll=10)
    svec, gvec = st[8], st[9]

    scores_ref[0] = svec
    gidx_ref[0] = gvec


def kernel(belive_map):
    B, S, H, W = belive_map.shape
    bs = B * S
    nprog = bs // _C
    xflat = belive_map.reshape(bs, H, W)
    raw_scores, raw_gidx = pl.pallas_call(
        _nms_kernel,
        grid=(nprog,),
        in_specs=[pl.BlockSpec((_C, H, W), lambda i: (i, 0, 0))],
        out_specs=[
            pl.BlockSpec((1, _C, 128), lambda i: (i, 0, 0)),
            pl.BlockSpec((1, _C, 128), lambda i: (i, 0, 0)),
        ],
        out_shape=[
            jax.ShapeDtypeStruct((nprog, _C, 128), jnp.float32),
            jax.ShapeDtypeStruct((nprog, _C, 128), jnp.int32),
        ],
        scratch_shapes=[pltpu.VMEM((_C, W, H), jnp.float32)],
        compiler_params=pltpu.CompilerParams(
            dimension_semantics=("arbitrary",)),
    )(xflat)

    scores_raw = raw_scores.reshape(bs, 128)[:, :_MAX_NUM_PEAKS]
    scores_raw = scores_raw.reshape(B, S, _MAX_NUM_PEAKS)
    g = raw_gidx.reshape(bs, 128)[:, :_MAX_NUM_PEAKS].reshape(B, S,
                                                              _MAX_NUM_PEAKS)
    valid = scores_raw > 0.0
    scores = jnp.where(valid, scores_raw, 0.0)
    rows = g // W
    cols = g % W
    seg = jnp.broadcast_to(jnp.arange(S, dtype=jnp.int32)[None, :, None],
                           (B, S, _MAX_NUM_PEAKS))
    skeletons = jnp.stack([seg, cols, rows], axis=-1)
    return skeletons, scores, valid


# speedup vs baseline: 1.5746x; 1.0701x over previous
import jax
import jax.numpy as jnp
from jax.experimental import pallas as pl
from jax.experimental.pallas import tpu as pltpu

_MIN_DISTANCE = 3
_THRESHOLD_REL = 0.01
_MAX_NUM_PEAKS = 100
_H = 512
_W = 512
_C = 10
_BIG = 1 << 30
_T = 3


def _pool7(padded):
    n = padded.shape[1] - 6
    t2 = jnp.maximum(padded[:, :n + 5], padded[:, 1:])
    t4 = jnp.maximum(t2[:, :n + 3], t2[:, 2:])
    return jnp.maximum(t4[:, :n], t4[:, 3:])


def _nms_kernel(x_ref, scores_ref, gidx_ref, candT_ref):
    x = x_ref[...]

    m = jnp.max(x, axis=(1, 2), keepdims=True)
    e = jnp.exp(x - m)
    s = jnp.sum(e, axis=(1, 2), keepdims=True)
    p = e / s

    zr = jnp.zeros((_C, _MIN_DISTANCE, _W), jnp.float32)
    pooled_v = _pool7(jnp.concatenate([zr, p, zr], axis=1))
    pT = jnp.swapaxes(p, 1, 2)
    pvT = jnp.swapaxes(pooled_v, 1, 2)
    zc = jnp.zeros((_C, _MIN_DISTANCE, _H), jnp.float32)
    pooledT = _pool7(jnp.concatenate([zc, pvT, zc], axis=1))

    thr_abs = 1.0 / (_H * _W) * 2.0
    mx = jnp.max(pT, axis=(1, 2), keepdims=True)
    maskT = (pooledT == pT) & (pT > thr_abs) & (pT > _THRESHOLD_REL * mx)
    candT = jnp.where(maskT, pT, jnp.float32(-1.0))
    candT_ref[...] = candT

    rowsT = jax.lax.broadcasted_iota(jnp.int32, (_C, _W, _H), 2)
    levels = []
    cT = candT
    for _ in range(_T):
        smax = jnp.max(cT, axis=2)
        sarg = jnp.min(jnp.where(cT == smax[:, :, None], rowsT, _BIG),
                       axis=2)
        cT = jnp.where(rowsT == sarg[:, :, None], jnp.float32(-3.0), cT)
        levels.append((smax, sarg))
    candT_ref[...] = cT
    (cmax, carg), (s1m, s1a), (s2m, s2a) = levels

    lane_w = jax.lax.broadcasted_iota(jnp.int32, (_C, _W), 1)
    lane_h = jax.lax.broadcasted_iota(jnp.int32, (_C, _H), 1)
    lane_k = jax.lax.broadcasted_iota(jnp.int32, (_C, 128), 1)

    def body(i, st):
        cmax, carg, s1m, s1a, s2m, s2a, svec, gvec = st
        mval = jnp.max(cmax, axis=1, keepdims=True)
        g = jnp.min(jnp.where(cmax == mval, carg * _W + lane_w, _BIG),
                    axis=1, keepdims=True)
        svec = jnp.where(lane_k == i, mval, svec)
        gvec = jnp.where(lane_k == i, g, gvec)
        hit = lane_w == g % _W
        cmax = jnp.where(hit, s1m, cmax)
        carg = jnp.where(hit, s1a, carg)
        s1m = jnp.where(hit, s2m, s1m)
        s1a = jnp.where(hit, s2a, s1a)
        rowsbuf = jnp.concatenate(
            [candT_ref[pl.ds(ch, 1), pl.ds(g[ch, 0] % _W, 1), :]
             .reshape(1, _H) for ch in range(_C)], axis=0)
        nm = jnp.max(rowsbuf, axis=1, keepdims=True)
        na = jnp.min(jnp.where(rowsbuf == nm, lane_h, _BIG),
                     axis=1, keepdims=True)
        rowsbuf = jnp.where(lane_h == na, jnp.float32(-3.0), rowsbuf)
        for ch in range(_C):
            candT_ref[pl.ds(ch, 1), pl.ds(g[ch, 0] % _W, 1), :] = (
                rowsbuf[ch:ch + 1].reshape(1, 1, _H))
        s2m = jnp.where(hit, nm, s2m)
        s2a = jnp.where(hit, na, s2a)
        return cmax, carg, s1m, s1a, s2m, s2a, svec, gvec

    svec0 = jnp.zeros((_C, 128), jnp.float32)
    gvec0 = jnp.zeros((_C, 128), jnp.int32)
    st = (cmax, carg, s1m, s1a, s2m, s2a, svec0, gvec0)
    st = jax.lax.fori_loop(0, _MAX_NUM_PEAKS, body, st, unroll=10)
    svec, gvec = st[6], st[7]

    scores_ref[0] = svec
    gidx_ref[0] = gvec


def kernel(belive_map):
    B, S, H, W = belive_map.shape
    bs = B * S
    nprog = bs // _C
    xflat = belive_map.reshape(bs, H, W)
    raw_scores, raw_gidx = pl.pallas_call(
        _nms_kernel,
        grid=(nprog,),
        in_specs=[pl.BlockSpec((_C, H, W), lambda i: (i, 0, 0))],
        out_specs=[
            pl.BlockSpec((1, _C, 128), lambda i: (i, 0, 0)),
            pl.BlockSpec((1, _C, 128), lambda i: (i, 0, 0)),
        ],
        out_shape=[
            jax.ShapeDtypeStruct((nprog, _C, 128), jnp.float32),
            jax.ShapeDtypeStruct((nprog, _C, 128), jnp.int32),
        ],
        scratch_shapes=[pltpu.VMEM((_C, W, H), jnp.float32)],
        compiler_params=pltpu.CompilerParams(
            dimension_semantics=("arbitrary",)),
    )(xflat)

    scores_raw = raw_scores.reshape(bs, 128)[:, :_MAX_NUM_PEAKS]
    scores_raw = scores_raw.reshape(B, S, _MAX_NUM_PEAKS)
    g = raw_gidx.reshape(bs, 128)[:, :_MAX_NUM_PEAKS].reshape(B, S,
                                                              _MAX_NUM_PEAKS)
    valid = scores_raw > 0.0
    scores = jnp.where(valid, scores_raw, 0.0)
    rows = g // W
    cols = g % W
    seg = jnp.broadcast_to(jnp.arange(S, dtype=jnp.int32)[None, :, None],
                           (B, S, _MAX_NUM_PEAKS))
    skeletons = jnp.stack([seg, cols, rows], axis=-1)
    return skeletons, scores, valid


# speedup vs baseline: 1.6266x; 1.0330x over previous
import jax
import jax.numpy as jnp
from jax.experimental import pallas as pl
from jax.experimental.pallas import tpu as pltpu

_MIN_DISTANCE = 3
_THRESHOLD_REL = 0.01
_MAX_NUM_PEAKS = 100
_H = 512
_W = 512
_C = 10
_BIG = 1 << 30
_T = 3


def _pool7(padded):
    n = padded.shape[1] - 6
    t2 = jnp.maximum(padded[:, :n + 5], padded[:, 1:])
    t4 = jnp.maximum(t2[:, :n + 3], t2[:, 2:])
    return jnp.maximum(t4[:, :n], t4[:, 3:])


def _nms_kernel(x_ref, scores_ref, gidx_ref, candT_ref):
    x = x_ref[...]

    m = jnp.max(x, axis=(1, 2), keepdims=True)
    e = jnp.exp(x - m)
    s = jnp.sum(e, axis=(1, 2), keepdims=True)
    p = e / s

    zr = jnp.zeros((_C, _MIN_DISTANCE, _W), jnp.float32)
    pooled_v = _pool7(jnp.concatenate([zr, p, zr], axis=1))
    pT = jnp.swapaxes(p, 1, 2)
    pvT = jnp.swapaxes(pooled_v, 1, 2)
    zc = jnp.zeros((_C, _MIN_DISTANCE, _H), jnp.float32)
    pooledT = _pool7(jnp.concatenate([zc, pvT, zc], axis=1))

    thr_abs = 1.0 / (_H * _W) * 2.0
    mx = jnp.max(pT, axis=(1, 2), keepdims=True)
    maskT = (pooledT == pT) & (pT > thr_abs) & (pT > _THRESHOLD_REL * mx)
    candT = jnp.where(maskT, pT, jnp.float32(-1.0))
    candT_ref[...] = candT

    rowsT = jax.lax.broadcasted_iota(jnp.int32, (_C, _W, _H), 2)
    levels = []
    cT = candT
    for _ in range(_T):
        smax = jnp.max(cT, axis=2)
        sarg = jnp.min(jnp.where(cT == smax[:, :, None], rowsT, _BIG),
                       axis=2)
        cT = jnp.where(rowsT == sarg[:, :, None], jnp.float32(-3.0), cT)
        levels.append((smax, sarg))
    candT_ref[...] = cT
    (cmax, carg), (s1m, s1a), (s2m, s2a) = levels

    lane_w = jax.lax.broadcasted_iota(jnp.int32, (_C, _W), 1)
    lane_h = jax.lax.broadcasted_iota(jnp.int32, (_C, _H), 1)
    lane_k = jax.lax.broadcasted_iota(jnp.int32, (_C, 128), 1)

    def body(i, st):
        cmax, carg, s1m, s1a, s2m, s2a, svec, gvec = st
        mval = jnp.max(cmax, axis=1, keepdims=True)
        g = jnp.min(jnp.where(cmax == mval, carg * _W + lane_w, _BIG),
                    axis=1, keepdims=True)
        svec = jnp.where(lane_k == i, mval, svec)
        gvec = jnp.where(lane_k == i, g, gvec)
        hit = lane_w == g % _W
        cmax = jnp.where(hit, s1m, cmax)
        carg = jnp.where(hit, s1a, carg)
        s1m = jnp.where(hit, s2m, s1m)
        s1a = jnp.where(hit, s2a, s1a)
        rowsbuf = jnp.concatenate(
            [candT_ref[pl.ds(ch, 1), pl.ds(g[ch, 0] % _W, 1), :]
             .reshape(1, _H) for ch in range(_C)], axis=0)
        nm = jnp.max(rowsbuf, axis=1, keepdims=True)
        na = jnp.min(jnp.where(rowsbuf == nm, lane_h, _BIG),
                     axis=1, keepdims=True)
        rowsbuf = jnp.where(lane_h == na, jnp.float32(-3.0), rowsbuf)
        for ch in range(_C):
            candT_ref[pl.ds(ch, 1), pl.ds(g[ch, 0] % _W, 1), :] = (
                rowsbuf[ch:ch + 1].reshape(1, 1, _H))
        s2m = jnp.where(hit, nm, s2m)
        s2a = jnp.where(hit, na, s2a)
        return cmax, carg, s1m, s1a, s2m, s2a, svec, gvec

    svec0 = jnp.zeros((_C, 128), jnp.float32)
    gvec0 = jnp.zeros((_C, 128), jnp.int32)
    st = (cmax, carg, s1m, s1a, s2m, s2a, svec0, gvec0)
    st = jax.lax.fori_loop(0, _MAX_NUM_PEAKS, body, st, unroll=25)
    svec, gvec = st[6], st[7]

    scores_ref[0] = svec
    gidx_ref[0] = gvec


def kernel(belive_map):
    B, S, H, W = belive_map.shape
    bs = B * S
    nprog = bs // _C
    xflat = belive_map.reshape(bs, H, W)
    raw_scores, raw_gidx = pl.pallas_call(
        _nms_kernel,
        grid=(nprog,),
        in_specs=[pl.BlockSpec((_C, H, W), lambda i: (i, 0, 0))],
        out_specs=[
            pl.BlockSpec((1, _C, 128), lambda i: (i, 0, 0)),
            pl.BlockSpec((1, _C, 128), lambda i: (i, 0, 0)),
        ],
        out_shape=[
            jax.ShapeDtypeStruct((nprog, _C, 128), jnp.float32),
            jax.ShapeDtypeStruct((nprog, _C, 128), jnp.int32),
        ],
        scratch_shapes=[pltpu.VMEM((_C, W, H), jnp.float32)],
        compiler_params=pltpu.CompilerParams(
            dimension_semantics=("arbitrary",)),
    )(xflat)

    scores_raw = raw_scores.reshape(bs, 128)[:, :_MAX_NUM_PEAKS]
    scores_raw = scores_raw.reshape(B, S, _MAX_NUM_PEAKS)
    g = raw_gidx.reshape(bs, 128)[:, :_MAX_NUM_PEAKS].reshape(B, S,
                                                              _MAX_NUM_PEAKS)
    valid = scores_raw > 0.0
    scores = jnp.where(valid, scores_raw, 0.0)
    rows = g // W
    cols = g % W
    seg = jnp.broadcast_to(jnp.arange(S, dtype=jnp.int32)[None, :, None],
                           (B, S, _MAX_NUM_PEAKS))
    skeletons = jnp.stack([seg, cols, rows], axis=-1)
    return skeletons, scores, valid
